# Initial kernel scaffold; baseline (speedup 1.0000x reference)
import jax
import jax.numpy as jnp
from jax.experimental import pallas as pl


def kernel(x, metal_feature, edge_attr, Wmlp, bmlp, eW1, eW2, bn_g, bn_b, sage_rel_W, sage_rel_b, sage_root_W, sage_root_b, pool_g, pool_b, Wq, Wk, Wv, out_W, out_b, batch, edge_index):
    raise NotImplementedError("write your pallas kernel here")


# speedup vs baseline: 1.0958x; 1.0958x over previous
import functools

import jax
import jax.numpy as jnp
from jax import lax
from jax.experimental import pallas as pl
from jax.experimental.pallas import tpu as pltpu

N = 10000; E = 160000; D = 256; B = 64; NMAX = 512; C = 16; L = 5; DK = 256
F32 = jnp.float32


def _mm(a, b):
    return lax.dot_general(a, b, (((a.ndim - 1,), (0,)), ((), ())),
                           precision=lax.Precision.DEFAULT,
                           preferred_element_type=F32)
_VMEM_BIG = pltpu.CompilerParams(vmem_limit_bytes=63 * 1024 * 1024)
_NB = 5
_BS = N // _NB


def _prep_body(x_ref, deg_ref, dis_ref, hh_ref):
    deg = deg_ref[...]
    dis = jnp.where(deg > 0.0, lax.rsqrt(jnp.maximum(deg, 1e-30)), 0.0)
    dis_ref[...] = dis
    hh_ref[...] = x_ref[...] * dis


def _prep(x, deg):
    return pl.pallas_call(
        _prep_body,
        out_shape=[jax.ShapeDtypeStruct((N, 1), F32),
                   jax.ShapeDtypeStruct((N, D), F32)],
        compiler_params=_VMEM_BIG,
    )(x, deg)


def _rowspec(w):
    return pl.BlockSpec((_BS, w), lambda i: (i, 0))


def _fullspec(r, c):
    return pl.BlockSpec((r, c), lambda i: (0, 0))


def _layer_z_body(h_ref, agg0_ref, b_ref, W1_ref, W2_ref,
                  z_ref, mu_ref, acc_ref):
    i = pl.program_id(0)

    @pl.when(i == 0)
    def _():
        acc_ref[...] = jnp.zeros_like(acc_ref)

    z = _mm(h_ref[...], W1_ref[...]) + _mm(agg0_ref[...], W2_ref[...]) + b_ref[...]
    z = jnp.maximum(z, 0.0)
    z_ref[...] = z
    acc_ref[...] += jnp.sum(z, axis=0, keepdims=True)

    @pl.when(i == _NB - 1)
    def _():
        mu_ref[...] = acc_ref[...] * (1.0 / N)


def _var_body(z_ref, mu_ref, var_ref, acc_ref):
    i = pl.program_id(0)

    @pl.when(i == 0)
    def _():
        acc_ref[...] = jnp.zeros_like(acc_ref)

    zc = z_ref[...] - mu_ref[...]
    acc_ref[...] += jnp.sum(zc * zc, axis=0, keepdims=True)

    @pl.when(i == _NB - 1)
    def _():
        var_ref[...] = acc_ref[...] * (1.0 / N)


def _bn_body(z_ref, mu_ref, var_ref, g_ref, bb_ref, h2_ref, *, last):
    z = ((z_ref[...] - mu_ref[...]) * lax.rsqrt(var_ref[...] + 1e-5)
         * g_ref[...] + bb_ref[...])
    if not last:
        z = jnp.maximum(z, 0.0)
    h2_ref[...] = z


def _layer(h, agg0, W1, W2, b, g, bb, last):
    z, mu = pl.pallas_call(
        _layer_z_body,
        grid=(_NB,),
        in_specs=[_rowspec(D), _rowspec(D), _fullspec(1, D),
                  _fullspec(D, D), _fullspec(D, D)],
        out_specs=[_rowspec(D), _fullspec(1, D)],
        out_shape=[jax.ShapeDtypeStruct((N, D), F32),
                   jax.ShapeDtypeStruct((1, D), F32)],
        scratch_shapes=[pltpu.VMEM((1, D), F32)],
    )(h, agg0, b, W1, W2)
    var = pl.pallas_call(
        _var_body,
        grid=(_NB,),
        in_specs=[_rowspec(D), _fullspec(1, D)],
        out_specs=_fullspec(1, D),
        out_shape=jax.ShapeDtypeStruct((1, D), F32),
        scratch_shapes=[pltpu.VMEM((1, D), F32)],
    )(z, mu)
    return pl.pallas_call(
        functools.partial(_bn_body, last=last),
        grid=(_NB,),
        in_specs=[_rowspec(D), _fullspec(1, D), _fullspec(1, D),
                  _fullspec(1, D), _fullspec(1, D)],
        out_specs=_rowspec(D),
        out_shape=jax.ShapeDtypeStruct((N, D), F32),
    )(z, mu, var, g, bb)


def _tail1_body(h5_ref, agg2_ref, deg2_ref, relW_ref, rootW_ref, bias_ref,
                pg_ref, pb_ref, ss_ref):
    a = agg2_ref[...] / jnp.maximum(deg2_ref[...], 1.0)
    s = _mm(a, relW_ref[...]) + _mm(h5_ref[...], rootW_ref[...]) + bias_ref[...]
    denom = float(B * NMAX)
    mu = jnp.sum(s, axis=0, keepdims=True) / denom
    sc = s - mu
    var = (jnp.sum(sc * sc, axis=0, keepdims=True)
           + float(B * NMAX - N) * mu * mu) / denom
    s = sc * lax.rsqrt(var + 1e-5) * pg_ref[...] + pb_ref[...]
    s = jnp.maximum(s, 0.0)
    m = jnp.max(s, axis=1, keepdims=True)
    e = jnp.exp(s - m)
    ss_ref[...] = e / jnp.sum(e, axis=1, keepdims=True)


def _tail1(h5, agg2, deg2, relW, rootW, bias, pg, pb):
    return pl.pallas_call(
        _tail1_body,
        out_shape=jax.ShapeDtypeStruct((N, C), F32),
        compiler_params=_VMEM_BIG,
    )(h5, agg2, deg2, relW, rootW, bias, pg, pb)


def _tail2_body(dh_ref, dss_ref, mf_ref, Wq_ref, Wk_ref, Wv_ref, outW_ref,
                outb_ref, out_ref):
    dh = dh_ref[0]
    dss = dss_ref[0]
    xp = lax.dot_general(dss, dh, (((0,), (0,)), ((), ())),
                         precision=lax.Precision.DEFAULT,
                         preferred_element_type=F32)
    q = _mm(mf_ref[0], Wq_ref[...])
    k = _mm(xp, Wk_ref[...])
    v = _mm(xp, Wv_ref[...])
    logits = lax.dot_general(q, k, (((1,), (1,)), ((), ())),
                             precision=lax.Precision.DEFAULT,
                             preferred_element_type=F32) * (DK ** -0.5)
    m = jnp.max(logits, axis=1, keepdims=True)
    e = jnp.exp(logits - m)
    attn = e / jnp.sum(e, axis=1, keepdims=True)
    hatt = jnp.maximum(_mm(attn, v), 0.0)
    out_ref[0] = _mm(hatt, outW_ref[...]) + outb_ref[...]


def _tail2(dh, dss, mf, Wq, Wk, Wv, outW, outb):
    return pl.pallas_call(
        _tail2_body,
        grid=(B,),
        in_specs=[
            pl.BlockSpec((1, NMAX, D), lambda b: (b, 0, 0)),
            pl.BlockSpec((1, NMAX, C), lambda b: (b, 0, 0)),
            pl.BlockSpec((1, 1, D), lambda b: (b, 0, 0)),
            pl.BlockSpec((D, DK), lambda b: (0, 0)),
            pl.BlockSpec((D, DK), lambda b: (0, 0)),
            pl.BlockSpec((D, DK), lambda b: (0, 0)),
            pl.BlockSpec((DK, 1), lambda b: (0, 0)),
            pl.BlockSpec((1, 1), lambda b: (0, 0)),
        ],
        out_specs=pl.BlockSpec((1, 1, 1), lambda b: (b, 0, 0)),
        out_shape=jax.ShapeDtypeStruct((B, 1, 1), F32),
    )(dh, dss, mf[:, None], Wq, Wk, Wv, outW, outb)


def _edge_counts(row, col, batch):
    deg = jnp.zeros((N,), F32).at[col].add(1.0)
    validf = (batch[row] == batch[col]).astype(F32)
    deg2 = jnp.zeros((N,), F32).at[row].add(validf)
    return deg, deg2, validf


def _edge_agg(src_idx, dst_idx, norm, table, ef):
    msg = norm[:, None] * (table[src_idx] + ef)
    return jnp.zeros((N, D), F32).at[dst_idx].add(msg)


def _dense_gather(idxd, h5p, ssp):
    return h5p[idxd], ssp[idxd]


def kernel(x, metal_feature, edge_attr, Wmlp, bmlp, eW1, eW2, bn_g, bn_b,
           sage_rel_W, sage_rel_b, sage_root_W, sage_root_b, pool_g, pool_b,
           Wq, Wk, Wv, out_W, out_b, batch, edge_index):
    row = edge_index[0]
    col = edge_index[1]

    deg, deg2, validf = _edge_counts(row, col, batch)
    dis1 = jnp.where(deg > 0, deg ** -0.5, 0.0)
    norm = dis1[row] * dis1[col]

    h = x
    for l in range(L):
        ef = edge_attr[:, 0:3] @ eW1[l] + edge_attr[:, 3:4] @ eW2[l]
        agg0 = _edge_agg(row, col, norm, h, ef)
        h = _layer(h, agg0, Wmlp[l, :D], Wmlp[l, D:],
                   bmlp[l][None], bn_g[l][None], bn_b[l][None],
                   last=(l == L - 1))

    dst2 = jnp.where(validf > 0.5, row, N)
    h5p = jnp.concatenate([h, jnp.zeros((1, D), F32)], axis=0)
    h5r = h5p.astype(jnp.bfloat16).astype(F32)
    agg2 = jnp.zeros((N + 1, D), F32).at[dst2].add(h5r[col])[:N]

    ss = _tail1(h, agg2, deg2[:, None], sage_rel_W, sage_root_W,
                (sage_rel_b + sage_root_b)[None], pool_g[None], pool_b[None])

    starts_all = jnp.searchsorted(batch, jnp.arange(B + 1, dtype=batch.dtype))
    counts = jnp.diff(starts_all)
    p = jnp.arange(NMAX, dtype=jnp.int32)[None, :]
    idxd = jnp.where(p < counts[:, None], starts_all[:B, None] + p, N)
    ssp = jnp.concatenate([ss, jnp.zeros((1, C), F32)], axis=0)
    dh, dss = _dense_gather(idxd, h5p, ssp)

    out = _tail2(dh, dss, metal_feature, Wq, Wk, Wv, out_W, out_b[None])
    return out[:, :, 0]


# speedup vs baseline: 1.5237x; 1.3904x over previous
import functools

import jax
import jax.numpy as jnp
from jax import lax
from jax.experimental import pallas as pl
from jax.experimental.pallas import tpu as pltpu
from jax.experimental.pallas import tpu_sc as plsc

N = 10000; E = 160000; D = 256; B = 64; NMAX = 512; C = 16; L = 5; DK = 256
F32 = jnp.float32


def _mm(a, b):
    return lax.dot_general(a, b, (((a.ndim - 1,), (0,)), ((), ())),
                           precision=lax.Precision.DEFAULT,
                           preferred_element_type=F32)
_VMEM_BIG = pltpu.CompilerParams(vmem_limit_bytes=63 * 1024 * 1024)
_NB = 5
_BS = N // _NB


def _rowspec(w):
    return pl.BlockSpec((_BS, w), lambda i: (i, 0))


def _fullspec(r, c):
    return pl.BlockSpec((r, c), lambda i: (0, 0))


def _layer_z_body(h_ref, aggA_ref, aggB_ref, b_ref, W1_ref, W2_ref,
                  z_ref, mu_ref, acc_ref):
    i = pl.program_id(0)

    @pl.when(i == 0)
    def _():
        acc_ref[...] = jnp.zeros_like(acc_ref)

    agg = jnp.concatenate([aggA_ref[...], aggB_ref[...]], axis=1)
    z = _mm(h_ref[...], W1_ref[...]) + _mm(agg, W2_ref[...]) + b_ref[...]
    z = jnp.maximum(z, 0.0)
    z_ref[...] = z
    acc_ref[...] += jnp.sum(z, axis=0, keepdims=True)

    @pl.when(i == _NB - 1)
    def _():
        mu_ref[...] = acc_ref[...] * (1.0 / N)


def _var_body(z_ref, mu_ref, var_ref, acc_ref):
    i = pl.program_id(0)

    @pl.when(i == 0)
    def _():
        acc_ref[...] = jnp.zeros_like(acc_ref)

    zc = z_ref[...] - mu_ref[...]
    acc_ref[...] += jnp.sum(zc * zc, axis=0, keepdims=True)

    @pl.when(i == _NB - 1)
    def _():
        var_ref[...] = acc_ref[...] * (1.0 / N)


def _bn_body(z_ref, mu_ref, var_ref, g_ref, bb_ref, h2_ref, hh_ref, *, last):
    z = ((z_ref[...] - mu_ref[...]) * lax.rsqrt(var_ref[...] + 1e-5)
         * g_ref[...] + bb_ref[...])
    if not last:
        z = jnp.maximum(z, 0.0)
    h2_ref[...] = z
    hh_ref[0] = z[:, :128]
    hh_ref[1] = z[:, 128:]


def _layer(h, aggh, W1, W2, b, g, bb, last):
    z, mu = pl.pallas_call(
        _layer_z_body,
        grid=(_NB,),
        in_specs=[_rowspec(D),
                  pl.BlockSpec((_BS, 128), lambda i: (i, 0)),
                  pl.BlockSpec((_BS, 128), lambda i: (_NB + i, 0)),
                  _fullspec(1, D), _fullspec(D, D), _fullspec(D, D)],
        out_specs=[_rowspec(D), _fullspec(1, D)],
        out_shape=[jax.ShapeDtypeStruct((N, D), F32),
                   jax.ShapeDtypeStruct((1, D), F32)],
        scratch_shapes=[pltpu.VMEM((1, D), F32)],
    )(h, aggh, aggh, b, W1, W2)
    var = pl.pallas_call(
        _var_body,
        grid=(_NB,),
        in_specs=[_rowspec(D), _fullspec(1, D)],
        out_specs=_fullspec(1, D),
        out_shape=jax.ShapeDtypeStruct((1, D), F32),
        scratch_shapes=[pltpu.VMEM((1, D), F32)],
    )(z, mu)
    return pl.pallas_call(
        functools.partial(_bn_body, last=last),
        grid=(_NB,),
        in_specs=[_rowspec(D), _fullspec(1, D), _fullspec(1, D),
                  _fullspec(1, D), _fullspec(1, D)],
        out_specs=[_rowspec(D),
                   pl.BlockSpec((2, _BS, 128), lambda i: (0, i, 0))],
        out_shape=[jax.ShapeDtypeStruct((N, D), F32),
                   jax.ShapeDtypeStruct((2, N, 128), F32)],
    )(z, mu, var, g, bb)


def _tail1_body(h5_ref, agg2_ref, deg2_ref, relW_ref, rootW_ref, bias_ref,
                pg_ref, pb_ref, ss_ref):
    a = agg2_ref[...] / jnp.maximum(deg2_ref[...], 1.0)
    s = _mm(a, relW_ref[...]) + _mm(h5_ref[...], rootW_ref[...]) + bias_ref[...]
    denom = float(B * NMAX)
    mu = jnp.sum(s, axis=0, keepdims=True) / denom
    sc = s - mu
    var = (jnp.sum(sc * sc, axis=0, keepdims=True)
           + float(B * NMAX - N) * mu * mu) / denom
    s = sc * lax.rsqrt(var + 1e-5) * pg_ref[...] + pb_ref[...]
    s = jnp.maximum(s, 0.0)
    m = jnp.max(s, axis=1, keepdims=True)
    e = jnp.exp(s - m)
    ss = e / jnp.sum(e, axis=1, keepdims=True)
    ss_ref[...] = jnp.concatenate([ss, jnp.zeros((N, 128 - C), F32)], axis=1)


def _tail1(h5, agg2, deg2, relW, rootW, bias, pg, pb):
    return pl.pallas_call(
        _tail1_body,
        out_shape=jax.ShapeDtypeStruct((N, 128), F32),
        compiler_params=_VMEM_BIG,
    )(h5, agg2, deg2, relW, rootW, bias, pg, pb)


def _tail2_body(dhA_ref, dhB_ref, dss_ref, xp_ref):
    dh = jnp.concatenate([dhA_ref[0], dhB_ref[0]], axis=1)
    dss = dss_ref[0][:, :C]
    xp_ref[0] = lax.dot_general(dss, dh, (((0,), (0,)), ((), ())),
                                precision=lax.Precision.DEFAULT,
                                preferred_element_type=F32)


def _tail2(dhA, dhB, dss):
    return pl.pallas_call(
        _tail2_body,
        grid=(B,),
        in_specs=[
            pl.BlockSpec((1, NMAX, 128), lambda b: (b, 0, 0)),
            pl.BlockSpec((1, NMAX, 128), lambda b: (b, 0, 0)),
            pl.BlockSpec((1, NMAX, 128), lambda b: (b, 0, 0)),
        ],
        out_specs=pl.BlockSpec((1, C, D), lambda b: (b, 0, 0)),
        out_shape=jax.ShapeDtypeStruct((B, C, D), F32),
    )(dhA, dhB, dss)


def _mesh():
    return plsc.VectorSubcoreMesh(core_axis_name="c", subcore_axis_name="s")
_CK = 80
_EPT = E // 16
_NCH = _EPT // _CK
_RPT = 632
_RLAST = N - 15 * _RPT


def _al(v):
    return pl.multiple_of(v, 8)


def _i16(x):
    return jnp.zeros((16,), jnp.int32) + x


def _sc_edge_agg(src_hbm, dst_hbm, scale_hbm, tab_hbm, ef_hbm, eord_hbm,
                 zeros_hbm, out_hbm, sidx, didx, nbuf, hbuf, vbuf, eidx, acc,
                 sem, *, use_ef):
    c = lax.axis_index("c")
    s = lax.axis_index("s")
    r0 = _al(s * _RPT)

    @pl.when(s < 15)
    def _():
        pltpu.sync_copy(zeros_hbm.at[pl.ds(r0, _RPT)],
                        acc.at[pl.ds(r0, _RPT)])

    @pl.when(s == 15)
    def _():
        pltpu.sync_copy(zeros_hbm.at[pl.ds(r0, _RLAST)],
                        acc.at[pl.ds(r0, _RLAST)])
    plsc.subcore_barrier()

    def chunk(t, _):
        base = _al(s * _EPT + t * _CK)
        pltpu.sync_copy(src_hbm.at[pl.ds(base, _CK)], sidx)
        pltpu.sync_copy(dst_hbm.at[pl.ds(base, _CK)], didx)
        pltpu.sync_copy(scale_hbm.at[pl.ds(base, _CK)], nbuf)

        def adj(i, _):
            sl = pl.ds(i * 16, 16)
            sidx[sl] = sidx[sl] + c * N
            return 0
        lax.fori_loop(0, _CK // 16, adj, 0)
        pltpu.async_copy(tab_hbm.at[sidx], hbuf, sem).wait()
        if use_ef:
            pltpu.sync_copy(eord_hbm.at[pl.ds(base, _CK)], eidx)

            def eadj(i, _):
                sl = pl.ds(i * 16, 16)
                eidx[sl] = eidx[sl] + c * E
                return 0
            lax.fori_loop(0, _CK // 16, eadj, 0)
            pltpu.async_copy(ef_hbm.at[eidx], vbuf, sem).wait()

        def edge(e, _):
            nv = nbuf[e]
            for j in range(8):
                sl = pl.ds(j * 16, 16)
                if use_ef:
                    vbuf[e, sl] = (hbuf[e, sl] + vbuf[e, sl]) * nv
                else:
                    vbuf[e, sl] = hbuf[e, sl] * nv
            return 0
        lax.fori_loop(0, _CK, edge, 0)
        pltpu.sync_copy(vbuf, acc.at[didx], add=True)
        return 0
    lax.fori_loop(0, _NCH, chunk, 0)
    plsc.subcore_barrier()

    @pl.when(s < 15)
    def _():
        pltpu.sync_copy(acc.at[pl.ds(r0, _RPT)],
                        out_hbm.at[pl.ds(_al(c * N + r0), _RPT)])

    @pl.when(s == 15)
    def _():
        pltpu.sync_copy(acc.at[pl.ds(r0, _RLAST)],
                        out_hbm.at[pl.ds(_al(c * N + r0), _RLAST)])


def _edge_agg_sc(src, dst, scale, tab2, ef2, eord, use_ef):
    zeros = jnp.zeros((N, 128), F32)
    kern = pl.kernel(
        out_type=jax.ShapeDtypeStruct((2 * N, 128), F32),
        mesh=_mesh(),
        scratch_types=[
            pltpu.VMEM((_CK,), jnp.int32),
            pltpu.VMEM((_CK,), jnp.int32),
            pltpu.VMEM((_CK, 16), F32),
            pltpu.VMEM((_CK, 128), F32),
            pltpu.VMEM((_CK, 128), F32),
            pltpu.VMEM((_CK,), jnp.int32),
            pltpu.VMEM_SHARED((N, 128), F32),
            pltpu.SemaphoreType.DMA,
        ],
    )(functools.partial(_sc_edge_agg, use_ef=use_ef))
    return kern(src, dst, scale, tab2, ef2, eord, zeros)


def _sc_counts(row_hbm, col_hbm, batch_hbm, zeros_hbm, degp_hbm, deg2p_hbm,
               valid_hbm, sidx, didx, bsrc, bdst, vbuf, obuf, dacc, d2acc,
               sem):
    c = lax.axis_index("c")
    s = lax.axis_index("s")

    @pl.when(s == 0)
    def _():
        pltpu.sync_copy(zeros_hbm, dacc)
        pltpu.sync_copy(zeros_hbm, d2acc)
    plsc.subcore_barrier()

    def chunk(t, _):
        base = _al(s * _EPT + t * _CK)
        pltpu.sync_copy(row_hbm.at[pl.ds(base, _CK)], sidx)
        pltpu.sync_copy(col_hbm.at[pl.ds(base, _CK)], didx)
        pltpu.async_copy(batch_hbm.at[sidx], bsrc, sem).wait()
        pltpu.async_copy(batch_hbm.at[didx], bdst, sem).wait()

        def cmp(i, _):
            sl = pl.ds(i * 16, 16)
            eq = bsrc[sl] == bdst[sl]
            vbuf[sl] = jnp.where(eq, 1.0, 0.0)
            obuf[sl] = jnp.zeros((16,), F32) + 1.0
            return 0
        lax.fori_loop(0, _CK // 16, cmp, 0)
        pltpu.sync_copy(obuf, dacc.at[didx], add=True)
        pltpu.sync_copy(vbuf, d2acc.at[sidx], add=True)
        pltpu.sync_copy(vbuf, valid_hbm.at[pl.ds(base, _CK)])
        return 0

    @pl.when(c == 0)
    def _():
        lax.fori_loop(0, _NCH, chunk, 0)
    plsc.subcore_barrier()

    @pl.when(s == 0)
    def _():
        pltpu.sync_copy(dacc, degp_hbm.at[c])
        pltpu.sync_copy(d2acc, deg2p_hbm.at[c])


def _edge_counts_sc(row, col, batch):
    degp, deg2p, validf = pl.kernel(
        out_type=[jax.ShapeDtypeStruct((2, N), F32),
                  jax.ShapeDtypeStruct((2, N), F32),
                  jax.ShapeDtypeStruct((E,), F32)],
        mesh=_mesh(),
        scratch_types=[
            pltpu.VMEM((_CK,), jnp.int32),
            pltpu.VMEM((_CK,), jnp.int32),
            pltpu.VMEM((_CK,), jnp.int32),
            pltpu.VMEM((_CK,), jnp.int32),
            pltpu.VMEM((_CK,), F32),
            pltpu.VMEM((_CK,), F32),
            pltpu.VMEM_SHARED((N,), F32),
            pltpu.VMEM_SHARED((N,), F32),
            pltpu.SemaphoreType.DMA,
        ],
    )(_sc_counts)(row, col, batch, jnp.zeros((N,), F32))
    return degp[0] + degp[1], deg2p[0] + deg2p[1], validf


def _sc_norm(row_hbm, col_hbm, dis_hbm, norm_hbm,
             sidx, didx, dsrc, ddst, vbuf, sem):
    c = lax.axis_index("c")
    s = lax.axis_index("s")
    def chunk(t, _):
        base = _al(s * _EPT + t * _CK)
        pltpu.sync_copy(row_hbm.at[pl.ds(base, _CK)], sidx)
        pltpu.sync_copy(col_hbm.at[pl.ds(base, _CK)], didx)
        pltpu.async_copy(dis_hbm.at[sidx], dsrc, sem).wait()
        pltpu.async_copy(dis_hbm.at[didx], ddst, sem).wait()

        def mul(i, _):
            sl = pl.ds(i * 16, 16)
            vbuf[sl] = dsrc[sl] * ddst[sl]
            return 0
        lax.fori_loop(0, _CK // 16, mul, 0)
        pltpu.sync_copy(vbuf, norm_hbm.at[pl.ds(base, _CK)])
        return 0

    @pl.when(c == 0)
    def _():
        lax.fori_loop(0, _NCH, chunk, 0)


def _edge_norm_sc(row, col, dis1):
    return pl.kernel(
        out_type=jax.ShapeDtypeStruct((E,), F32),
        mesh=_mesh(),
        scratch_types=[
            pltpu.VMEM((_CK,), jnp.int32),
            pltpu.VMEM((_CK,), jnp.int32),
            pltpu.VMEM((_CK,), F32),
            pltpu.VMEM((_CK,), F32),
            pltpu.VMEM((_CK,), F32),
            pltpu.SemaphoreType.DMA,
        ],
    )(_sc_norm)(row, col, dis1)


_DG = B * NMAX
_DPT = _DG // 16
_SPT = _DG // 32


def _sc_dense(idx_hbm, tab_hbm, ss_hbm, dh_hbm, dss_hbm,
              ibuf, hbuf, sbuf, sem):
    c = lax.axis_index("c")
    s = lax.axis_index("s")

    def hchunk(t, _):
        base = _al(s * _DPT + t * 128)
        pltpu.sync_copy(idx_hbm.at[pl.ds(base, 128)], ibuf)

        def adj(i, _):
            sl = pl.ds(i * 16, 16)
            ibuf[sl] = ibuf[sl] + c * (N + 1)
            return 0
        lax.fori_loop(0, 8, adj, 0)
        pltpu.async_copy(tab_hbm.at[ibuf], hbuf, sem).wait()
        pltpu.sync_copy(hbuf, dh_hbm.at[pl.ds(_al(c * _DG + base), 128)])
        return 0
    lax.fori_loop(0, _DPT // 128, hchunk, 0)

    def schunk(t, _):
        base = _al((s * 2 + c) * _SPT + t * 128)
        pltpu.sync_copy(idx_hbm.at[pl.ds(base, 128)], ibuf)
        pltpu.async_copy(ss_hbm.at[ibuf], sbuf, sem).wait()
        pltpu.sync_copy(sbuf, dss_hbm.at[pl.ds(base, 128)])
        return 0
    lax.fori_loop(0, _SPT // 128, schunk, 0)


def _dense_gather_sc(idxd, h5p2, ssp):
    return pl.kernel(
        out_type=[jax.ShapeDtypeStruct((2 * _DG, 128), F32),
                  jax.ShapeDtypeStruct((_DG, 128), F32)],
        mesh=_mesh(),
        scratch_types=[
            pltpu.VMEM((128,), jnp.int32),
            pltpu.VMEM((128, 128), F32),
            pltpu.VMEM((128, 128), F32),
            pltpu.SemaphoreType.DMA,
        ],
    )(_sc_dense)(idxd, h5p2, ssp)


def kernel(x, metal_feature, edge_attr, Wmlp, bmlp, eW1, eW2, bn_g, bn_b,
           sage_rel_W, sage_rel_b, sage_root_W, sage_root_b, pool_g, pool_b,
           Wq, Wk, Wv, out_W, out_b, batch, edge_index):
    row = edge_index[0]
    col = edge_index[1]

    deg, deg2, validf = _edge_counts_sc(row, col, batch)
    dis1 = jnp.where(deg > 0, deg ** -0.5, 0.0)
    norm = _edge_norm_sc(row, col, dis1)

    eord = jnp.argsort(col, stable=True).astype(jnp.int32)
    row_s = row[eord]
    col_s = col[eord]
    norm16_s = jnp.broadcast_to(norm[eord][:, None], (E, 16))

    h = x
    hh = jnp.concatenate([x[:, :128], x[:, 128:]], axis=0)
    for l in range(L):
        ef = edge_attr[:, 0:3] @ eW1[l] + edge_attr[:, 3:4] @ eW2[l]
        ef2 = jnp.concatenate([ef[:, :128], ef[:, 128:]], axis=0)
        aggh = _edge_agg_sc(row_s, col_s, norm16_s, hh, ef2, eord, use_ef=True)
        h, hh3 = _layer(h, aggh, Wmlp[l, :D], Wmlp[l, D:],
                        bmlp[l][None], bn_g[l][None], bn_b[l][None],
                        last=(l == L - 1))
        hh = hh3.reshape(2 * N, 128)

    hhr = hh.astype(jnp.bfloat16).astype(F32)
    valid16 = jnp.broadcast_to(validf[:, None], (E, 16))
    agg2h = _edge_agg_sc(col, row, valid16, hhr, hhr, row, use_ef=False)
    agg2 = jnp.concatenate([agg2h[:N], agg2h[N:]], axis=1)

    ss = _tail1(h, agg2, deg2[:, None], sage_rel_W, sage_root_W,
                (sage_rel_b + sage_root_b)[None], pool_g[None], pool_b[None])

    starts_all = jnp.searchsorted(batch, jnp.arange(B + 1, dtype=batch.dtype))
    counts = jnp.diff(starts_all)
    p = jnp.arange(NMAX, dtype=jnp.int32)[None, :]
    idxd = jnp.where(p < counts[:, None], starts_all[:B, None] + p,
                     N).reshape(-1).astype(jnp.int32)
    zrow = jnp.zeros((1, 128), F32)
    h5p2 = jnp.concatenate([hh[:N], zrow, hh[N:], zrow], axis=0)
    ssp = jnp.concatenate([ss, jnp.zeros((1, 128), F32)], axis=0)
    dhh, dss = _dense_gather_sc(idxd, h5p2, ssp)

    x_pool = _tail2(dhh[:_DG].reshape(B, NMAX, 128),
                    dhh[_DG:].reshape(B, NMAX, 128),
                    dss.reshape(B, NMAX, 128))
    Q = metal_feature[:, None, :] @ Wq
    K_ = x_pool @ Wk
    V = x_pool @ Wv
    attn = jax.nn.softmax(jnp.einsum('bqd,bkd->bqk', Q, K_)
                          / jnp.sqrt(float(DK)), axis=-1)
    hatt = jax.nn.relu(jnp.einsum('bqk,bkd->bqd', attn, V))[:, 0, :]
    return hatt @ out_W + out_b


# speedup vs baseline: 1.6144x; 1.0595x over previous
import functools

import jax
import jax.numpy as jnp
from jax import lax
from jax.experimental import pallas as pl
from jax.experimental.pallas import tpu as pltpu
from jax.experimental.pallas import tpu_sc as plsc

N = 10000; E = 160000; D = 256; B = 64; NMAX = 512; C = 16; L = 5; DK = 256
F32 = jnp.float32


def _mm(a, b):
    return lax.dot_general(a, b, (((a.ndim - 1,), (0,)), ((), ())),
                           precision=lax.Precision.DEFAULT,
                           preferred_element_type=F32)
_VMEM_BIG = pltpu.CompilerParams(vmem_limit_bytes=63 * 1024 * 1024)
_NB = 5
_BS = N // _NB


def _rowspec(w):
    return pl.BlockSpec((_BS, w), lambda i: (i, 0))


def _fullspec(r, c):
    return pl.BlockSpec((r, c), lambda i: (0, 0))


def _layer_z_body(h_ref, aggA_ref, aggB_ref, b_ref, W1_ref, W2_ref,
                  z_ref, mu_ref, acc_ref):
    i = pl.program_id(0)

    @pl.when(i == 0)
    def _():
        acc_ref[...] = jnp.zeros_like(acc_ref)

    agg = jnp.concatenate([aggA_ref[...], aggB_ref[...]], axis=1)
    z = _mm(h_ref[...], W1_ref[...]) + _mm(agg, W2_ref[...]) + b_ref[...]
    z = jnp.maximum(z, 0.0)
    z_ref[...] = z
    acc_ref[...] += jnp.sum(z, axis=0, keepdims=True)

    @pl.when(i == _NB - 1)
    def _():
        mu_ref[...] = acc_ref[...] * (1.0 / N)


def _var_body(z_ref, mu_ref, var_ref, acc_ref):
    i = pl.program_id(0)

    @pl.when(i == 0)
    def _():
        acc_ref[...] = jnp.zeros_like(acc_ref)

    zc = z_ref[...] - mu_ref[...]
    acc_ref[...] += jnp.sum(zc * zc, axis=0, keepdims=True)

    @pl.when(i == _NB - 1)
    def _():
        var_ref[...] = acc_ref[...] * (1.0 / N)


def _bn_body(z_ref, mu_ref, var_ref, g_ref, bb_ref, h2_ref, hh_ref, *, last):
    z = ((z_ref[...] - mu_ref[...]) * lax.rsqrt(var_ref[...] + 1e-5)
         * g_ref[...] + bb_ref[...])
    if not last:
        z = jnp.maximum(z, 0.0)
    h2_ref[...] = z
    hh_ref[0] = z[:, :128]
    hh_ref[1] = z[:, 128:]


def _layer(h, aggh, W1, W2, b, g, bb, last):
    z, mu = pl.pallas_call(
        _layer_z_body,
        grid=(_NB,),
        in_specs=[_rowspec(D),
                  pl.BlockSpec((_BS, 128), lambda i: (i, 0)),
                  pl.BlockSpec((_BS, 128), lambda i: (_NB + i, 0)),
                  _fullspec(1, D), _fullspec(D, D), _fullspec(D, D)],
        out_specs=[_rowspec(D), _fullspec(1, D)],
        out_shape=[jax.ShapeDtypeStruct((N, D), F32),
                   jax.ShapeDtypeStruct((1, D), F32)],
        scratch_shapes=[pltpu.VMEM((1, D), F32)],
    )(h, aggh, aggh, b, W1, W2)
    var = pl.pallas_call(
        _var_body,
        grid=(_NB,),
        in_specs=[_rowspec(D), _fullspec(1, D)],
        out_specs=_fullspec(1, D),
        out_shape=jax.ShapeDtypeStruct((1, D), F32),
        scratch_shapes=[pltpu.VMEM((1, D), F32)],
    )(z, mu)
    return pl.pallas_call(
        functools.partial(_bn_body, last=last),
        grid=(_NB,),
        in_specs=[_rowspec(D), _fullspec(1, D), _fullspec(1, D),
                  _fullspec(1, D), _fullspec(1, D)],
        out_specs=[_rowspec(D),
                   pl.BlockSpec((2, _BS, 128), lambda i: (0, i, 0))],
        out_shape=[jax.ShapeDtypeStruct((N, D), F32),
                   jax.ShapeDtypeStruct((2, N, 128), F32)],
    )(z, mu, var, g, bb)


def _tail1_body(h5_ref, agg2_ref, deg2_ref, relW_ref, rootW_ref, bias_ref,
                pg_ref, pb_ref, ss_ref):
    a = agg2_ref[...] / jnp.maximum(deg2_ref[...], 1.0)
    s = _mm(a, relW_ref[...]) + _mm(h5_ref[...], rootW_ref[...]) + bias_ref[...]
    denom = float(B * NMAX)
    mu = jnp.sum(s, axis=0, keepdims=True) / denom
    sc = s - mu
    var = (jnp.sum(sc * sc, axis=0, keepdims=True)
           + float(B * NMAX - N) * mu * mu) / denom
    s = sc * lax.rsqrt(var + 1e-5) * pg_ref[...] + pb_ref[...]
    s = jnp.maximum(s, 0.0)
    m = jnp.max(s, axis=1, keepdims=True)
    e = jnp.exp(s - m)
    ss = e / jnp.sum(e, axis=1, keepdims=True)
    ss_ref[...] = jnp.concatenate([ss, jnp.zeros((N, 128 - C), F32)], axis=1)


def _tail1(h5, agg2, deg2, relW, rootW, bias, pg, pb):
    return pl.pallas_call(
        _tail1_body,
        out_shape=jax.ShapeDtypeStruct((N, 128), F32),
        compiler_params=_VMEM_BIG,
    )(h5, agg2, deg2, relW, rootW, bias, pg, pb)


def _tail2_body(dhA_ref, dhB_ref, dss_ref, xp_ref):
    dh = jnp.concatenate([dhA_ref[0], dhB_ref[0]], axis=1)
    dss = dss_ref[0][:, :C]
    xp_ref[0] = lax.dot_general(dss, dh, (((0,), (0,)), ((), ())),
                                precision=lax.Precision.DEFAULT,
                                preferred_element_type=F32)


def _tail2(dhA, dhB, dss):
    return pl.pallas_call(
        _tail2_body,
        grid=(B,),
        in_specs=[
            pl.BlockSpec((1, NMAX, 128), lambda b: (b, 0, 0)),
            pl.BlockSpec((1, NMAX, 128), lambda b: (b, 0, 0)),
            pl.BlockSpec((1, NMAX, 128), lambda b: (b, 0, 0)),
        ],
        out_specs=pl.BlockSpec((1, C, D), lambda b: (b, 0, 0)),
        out_shape=jax.ShapeDtypeStruct((B, C, D), F32),
    )(dhA, dhB, dss)


def _mesh():
    return plsc.VectorSubcoreMesh(core_axis_name="c", subcore_axis_name="s")
_CK = 80
_EPT = E // 16
_NCH = _EPT // _CK
_RPT = 632
_RLAST = N - 15 * _RPT


def _al(v):
    return pl.multiple_of(v, 8)


def _i16(x):
    return jnp.zeros((16,), jnp.int32) + x


def _sc_edge_agg(src_hbm, dst_hbm, scale_hbm, tab_hbm, ef_hbm, eord_hbm,
                 zeros_hbm, out_hbm, sidx, didx, nbuf, hbuf, vbuf, eidx, acc,
                 sem, *, use_ef):
    c = lax.axis_index("c")
    s = lax.axis_index("s")
    r0 = _al(s * _RPT)

    @pl.when(s < 15)
    def _():
        pltpu.sync_copy(zeros_hbm.at[pl.ds(r0, _RPT)],
                        acc.at[pl.ds(r0, _RPT)])

    @pl.when(s == 15)
    def _():
        pltpu.sync_copy(zeros_hbm.at[pl.ds(r0, _RLAST)],
                        acc.at[pl.ds(r0, _RLAST)])
    plsc.subcore_barrier()

    def chunk(t, _):
        base = _al(s * _EPT + t * _CK)
        pltpu.sync_copy(src_hbm.at[pl.ds(base, _CK)], sidx)
        pltpu.sync_copy(dst_hbm.at[pl.ds(base, _CK)], didx)
        pltpu.sync_copy(scale_hbm.at[pl.ds(base, _CK)], nbuf)

        def adj(i, _):
            sl = pl.ds(i * 16, 16)
            sidx[sl] = sidx[sl] + c * N
            return 0
        lax.fori_loop(0, _CK // 16, adj, 0)
        if use_ef:
            pltpu.sync_copy(eord_hbm.at[pl.ds(base, _CK)], eidx)

            def eadj(i, _):
                sl = pl.ds(i * 16, 16)
                eidx[sl] = eidx[sl] + c * E
                return 0
            lax.fori_loop(0, _CK // 16, eadj, 0)
            cp1 = pltpu.async_copy(tab_hbm.at[sidx], hbuf, sem)
            cp2 = pltpu.async_copy(ef_hbm.at[eidx], vbuf, sem)
            cp1.wait()
            cp2.wait()
        else:
            pltpu.async_copy(tab_hbm.at[sidx], hbuf, sem).wait()

        def edge(e, _):
            nv = nbuf[e]
            for j in range(8):
                sl = pl.ds(j * 16, 16)
                if use_ef:
                    vbuf[e, sl] = (hbuf[e, sl] + vbuf[e, sl]) * nv
                else:
                    vbuf[e, sl] = hbuf[e, sl] * nv
            return 0
        lax.fori_loop(0, _CK, edge, 0)
        pltpu.sync_copy(vbuf, acc.at[didx], add=True)
        return 0
    lax.fori_loop(0, _NCH, chunk, 0)
    plsc.subcore_barrier()

    @pl.when(s < 15)
    def _():
        pltpu.sync_copy(acc.at[pl.ds(r0, _RPT)],
                        out_hbm.at[pl.ds(_al(c * N + r0), _RPT)])

    @pl.when(s == 15)
    def _():
        pltpu.sync_copy(acc.at[pl.ds(r0, _RLAST)],
                        out_hbm.at[pl.ds(_al(c * N + r0), _RLAST)])


def _edge_agg_sc(src, dst, scale, tab2, ef2, eord, use_ef):
    zeros = jnp.zeros((N, 128), F32)
    kern = pl.kernel(
        out_type=jax.ShapeDtypeStruct((2 * N, 128), F32),
        mesh=_mesh(),
        scratch_types=[
            pltpu.VMEM((_CK,), jnp.int32),
            pltpu.VMEM((_CK,), jnp.int32),
            pltpu.VMEM((_CK, 16), F32),
            pltpu.VMEM((_CK, 128), F32),
            pltpu.VMEM((_CK, 128), F32),
            pltpu.VMEM((_CK,), jnp.int32),
            pltpu.VMEM_SHARED((N, 128), F32),
            pltpu.SemaphoreType.DMA,
        ],
    )(functools.partial(_sc_edge_agg, use_ef=use_ef))
    return kern(src, dst, scale, tab2, ef2, eord, zeros)


def _sc_counts(row_hbm, col_hbm, batch_hbm, zeros_hbm, degp_hbm, deg2p_hbm,
               valid_hbm, sidx, didx, bsrc, bdst, vbuf, obuf, dacc, d2acc,
               sem):
    c = lax.axis_index("c")
    s = lax.axis_index("s")

    @pl.when(s == 0)
    def _():
        pltpu.sync_copy(zeros_hbm, dacc)
        pltpu.sync_copy(zeros_hbm, d2acc)
    plsc.subcore_barrier()

    def chunk(t, _):
        base = _al(s * _EPT + t * _CK)
        pltpu.sync_copy(row_hbm.at[pl.ds(base, _CK)], sidx)
        pltpu.sync_copy(col_hbm.at[pl.ds(base, _CK)], didx)
        pltpu.async_copy(batch_hbm.at[sidx], bsrc, sem).wait()
        pltpu.async_copy(batch_hbm.at[didx], bdst, sem).wait()

        def cmp(i, _):
            sl = pl.ds(i * 16, 16)
            eq = bsrc[sl] == bdst[sl]
            vbuf[sl] = jnp.where(eq, 1.0, 0.0)
            obuf[sl] = jnp.zeros((16,), F32) + 1.0
            return 0
        lax.fori_loop(0, _CK // 16, cmp, 0)
        pltpu.sync_copy(obuf, dacc.at[didx], add=True)
        pltpu.sync_copy(vbuf, d2acc.at[sidx], add=True)
        pltpu.sync_copy(vbuf, valid_hbm.at[pl.ds(base, _CK)])
        return 0

    @pl.when(c == 0)
    def _():
        lax.fori_loop(0, _NCH, chunk, 0)
    plsc.subcore_barrier()

    @pl.when(s == 0)
    def _():
        pltpu.sync_copy(dacc, degp_hbm.at[c])
        pltpu.sync_copy(d2acc, deg2p_hbm.at[c])


def _edge_counts_sc(row, col, batch):
    degp, deg2p, validf = pl.kernel(
        out_type=[jax.ShapeDtypeStruct((2, N), F32),
                  jax.ShapeDtypeStruct((2, N), F32),
                  jax.ShapeDtypeStruct((E,), F32)],
        mesh=_mesh(),
        scratch_types=[
            pltpu.VMEM((_CK,), jnp.int32),
            pltpu.VMEM((_CK,), jnp.int32),
            pltpu.VMEM((_CK,), jnp.int32),
            pltpu.VMEM((_CK,), jnp.int32),
            pltpu.VMEM((_CK,), F32),
            pltpu.VMEM((_CK,), F32),
            pltpu.VMEM_SHARED((N,), F32),
            pltpu.VMEM_SHARED((N,), F32),
            pltpu.SemaphoreType.DMA,
        ],
    )(_sc_counts)(row, col, batch, jnp.zeros((N,), F32))
    return degp[0] + degp[1], deg2p[0] + deg2p[1], validf


def _sc_norm(row_hbm, col_hbm, dis_hbm, norm_hbm,
             sidx, didx, dsrc, ddst, vbuf, sem):
    c = lax.axis_index("c")
    s = lax.axis_index("s")
    def chunk(t, _):
        base = _al(s * _EPT + t * _CK)
        pltpu.sync_copy(row_hbm.at[pl.ds(base, _CK)], sidx)
        pltpu.sync_copy(col_hbm.at[pl.ds(base, _CK)], didx)
        pltpu.async_copy(dis_hbm.at[sidx], dsrc, sem).wait()
        pltpu.async_copy(dis_hbm.at[didx], ddst, sem).wait()

        def mul(i, _):
            sl = pl.ds(i * 16, 16)
            vbuf[sl] = dsrc[sl] * ddst[sl]
            return 0
        lax.fori_loop(0, _CK // 16, mul, 0)
        pltpu.sync_copy(vbuf, norm_hbm.at[pl.ds(base, _CK)])
        return 0

    @pl.when(c == 0)
    def _():
        lax.fori_loop(0, _NCH, chunk, 0)


def _edge_norm_sc(row, col, dis1):
    return pl.kernel(
        out_type=jax.ShapeDtypeStruct((E,), F32),
        mesh=_mesh(),
        scratch_types=[
            pltpu.VMEM((_CK,), jnp.int32),
            pltpu.VMEM((_CK,), jnp.int32),
            pltpu.VMEM((_CK,), F32),
            pltpu.VMEM((_CK,), F32),
            pltpu.VMEM((_CK,), F32),
            pltpu.SemaphoreType.DMA,
        ],
    )(_sc_norm)(row, col, dis1)


_DG = B * NMAX
_DPT = _DG // 16
_SPT = _DG // 32


def _sc_dense(idx_hbm, tab_hbm, ss_hbm, dh_hbm, dss_hbm,
              ibuf, hbuf, sbuf, sem):
    c = lax.axis_index("c")
    s = lax.axis_index("s")

    def hchunk(t, _):
        base = _al(s * _DPT + t * 128)
        pltpu.sync_copy(idx_hbm.at[pl.ds(base, 128)], ibuf)

        def adj(i, _):
            sl = pl.ds(i * 16, 16)
            ibuf[sl] = ibuf[sl] + c * (N + 1)
            return 0
        lax.fori_loop(0, 8, adj, 0)
        pltpu.async_copy(tab_hbm.at[ibuf], hbuf, sem).wait()
        pltpu.sync_copy(hbuf, dh_hbm.at[pl.ds(_al(c * _DG + base), 128)])
        return 0
    lax.fori_loop(0, _DPT // 128, hchunk, 0)

    def schunk(t, _):
        base = _al((s * 2 + c) * _SPT + t * 128)
        pltpu.sync_copy(idx_hbm.at[pl.ds(base, 128)], ibuf)
        pltpu.async_copy(ss_hbm.at[ibuf], sbuf, sem).wait()
        pltpu.sync_copy(sbuf, dss_hbm.at[pl.ds(base, 128)])
        return 0
    lax.fori_loop(0, _SPT // 128, schunk, 0)


def _dense_gather_sc(idxd, h5p2, ssp):
    return pl.kernel(
        out_type=[jax.ShapeDtypeStruct((2 * _DG, 128), F32),
                  jax.ShapeDtypeStruct((_DG, 128), F32)],
        mesh=_mesh(),
        scratch_types=[
            pltpu.VMEM((128,), jnp.int32),
            pltpu.VMEM((128, 128), F32),
            pltpu.VMEM((128, 128), F32),
            pltpu.SemaphoreType.DMA,
        ],
    )(_sc_dense)(idxd, h5p2, ssp)


def kernel(x, metal_feature, edge_attr, Wmlp, bmlp, eW1, eW2, bn_g, bn_b,
           sage_rel_W, sage_rel_b, sage_root_W, sage_root_b, pool_g, pool_b,
           Wq, Wk, Wv, out_W, out_b, batch, edge_index):
    row = edge_index[0]
    col = edge_index[1]

    deg, deg2, validf = _edge_counts_sc(row, col, batch)
    dis1 = jnp.where(deg > 0, deg ** -0.5, 0.0)
    norm = _edge_norm_sc(row, col, dis1)

    eord = jnp.argsort(col, stable=True).astype(jnp.int32)
    row_s = row[eord]
    col_s = col[eord]
    norm16_s = jnp.broadcast_to(norm[eord][:, None], (E, 16))

    h = x
    hh = jnp.concatenate([x[:, :128], x[:, 128:]], axis=0)
    for l in range(L):
        ef = edge_attr[:, 0:3] @ eW1[l] + edge_attr[:, 3:4] @ eW2[l]
        ef2 = jnp.concatenate([ef[:, :128], ef[:, 128:]], axis=0)
        aggh = _edge_agg_sc(row_s, col_s, norm16_s, hh, ef2, eord, use_ef=True)
        h, hh3 = _layer(h, aggh, Wmlp[l, :D], Wmlp[l, D:],
                        bmlp[l][None], bn_g[l][None], bn_b[l][None],
                        last=(l == L - 1))
        hh = hh3.reshape(2 * N, 128)

    hhr = hh.astype(jnp.bfloat16).astype(F32)
    valid16 = jnp.broadcast_to(validf[:, None], (E, 16))
    agg2h = _edge_agg_sc(col, row, valid16, hhr, hhr, row, use_ef=False)
    agg2 = jnp.concatenate([agg2h[:N], agg2h[N:]], axis=1)

    ss = _tail1(h, agg2, deg2[:, None], sage_rel_W, sage_root_W,
                (sage_rel_b + sage_root_b)[None], pool_g[None], pool_b[None])

    starts_all = jnp.searchsorted(batch, jnp.arange(B + 1, dtype=batch.dtype))
    counts = jnp.diff(starts_all)
    p = jnp.arange(NMAX, dtype=jnp.int32)[None, :]
    idxd = jnp.where(p < counts[:, None], starts_all[:B, None] + p,
                     N).reshape(-1).astype(jnp.int32)
    zrow = jnp.zeros((1, 128), F32)
    h5p2 = jnp.concatenate([hh[:N], zrow, hh[N:], zrow], axis=0)
    ssp = jnp.concatenate([ss, jnp.zeros((1, 128), F32)], axis=0)
    dhh, dss = _dense_gather_sc(idxd, h5p2, ssp)

    x_pool = _tail2(dhh[:_DG].reshape(B, NMAX, 128),
                    dhh[_DG:].reshape(B, NMAX, 128),
                    dss.reshape(B, NMAX, 128))
    Q = metal_feature[:, None, :] @ Wq
    K_ = x_pool @ Wk
    V = x_pool @ Wv
    attn = jax.nn.softmax(jnp.einsum('bqd,bkd->bqk', Q, K_)
                          / jnp.sqrt(float(DK)), axis=-1)
    hatt = jax.nn.relu(jnp.einsum('bqk,bkd->bqd', attn, V))[:, 0, :]
    return hatt @ out_W + out_b
